# baseline (speedup 1.0000x reference)
import jax
import jax.numpy as jnp
from jax import lax
from jax.experimental import pallas as pl
from jax.experimental.pallas import tpu as pltpu
from jax.experimental.pallas import tpu_sc as plsc

R = 1000000
D = 32
B = 16384
RPB = 128 // D

NC = 2
NS = 16
NW = NC * NS
BPW = B // NW
WV = 128
NWAVE = BPW // WV

TCG = 8
TCB = B // TCG


def _gather_body(users_h, items_h, ut_h, it_h, ub_h, ib_h,
                 u0, u1, u2, u3, i0, i1, i2, i3,
                 gua, gia, gub, gib, sema, semb):
    wid = lax.axis_index("s") * NC + lax.axis_index("c")
    base = wid * BPW

    uq = [u0, u1, u2, u3]
    iq = [i0, i1, i2, i3]
    for k in range(NWAVE):
        pltpu.sync_copy(users_h.at[pl.ds(base + k * WV, WV)], uq[k])
        pltpu.sync_copy(items_h.at[pl.ds(base + k * WV, WV)], iq[k])
        for c in range(WV // 16):
            j = pl.ds(c * 16, 16)
            uq[k][j] = lax.shift_right_logical(uq[k][j], 2)
            iq[k][j] = lax.shift_right_logical(iq[k][j], 2)

    gu = [gua, gub]
    gi = [gia, gib]
    sems = [sema, semb]
    cps = [None] * NWAVE
    for k in range(2):
        cps[k] = (pltpu.async_copy(ut_h.at[uq[k]], gu[k], sems[k]),
                  pltpu.async_copy(it_h.at[iq[k]], gi[k], sems[k]))
    for k in range(NWAVE):
        p = k & 1
        cu, ci = cps[k]
        cu.wait()
        ci.wait()
        pltpu.sync_copy(gu[p], ub_h.at[pl.ds(base + k * WV, WV)])
        pltpu.sync_copy(gi[p], ib_h.at[pl.ds(base + k * WV, WV)])
        if k + 2 < NWAVE:
            cps[k + 2] = (pltpu.async_copy(ut_h.at[uq[k + 2]], gu[p], sems[p]),
                          pltpu.async_copy(it_h.at[iq[k + 2]], gi[p], sems[p]))


def _dense_body(ub_ref, ib_ref, uq_ref, iq_ref, w_ref, b_ref, o_ref):
    grp = lax.broadcasted_iota(jnp.int32, (TCB, RPB * D), 1) >> 5
    um = (grp == (uq_ref[...] & 3)).astype(jnp.float32)
    im = (grp == (iq_ref[...] & 3)).astype(jnp.float32)
    tu = ub_ref[...] * um
    ti = ib_ref[...] * im
    usel = tu[:, 0:32] + tu[:, 32:64] + tu[:, 64:96] + tu[:, 96:128]
    isel = ti[:, 0:32] + ti[:, 32:64] + ti[:, 64:96] + ti[:, 96:128]
    s = jnp.sum(usel * isel * w_ref[...], axis=1, keepdims=True) + b_ref[...]
    o_ref[...] = jax.nn.sigmoid(s)


@jax.jit
def _gmf(users, items, ut, it, users2, items2, wrow, brow):
    mesh = plsc.VectorSubcoreMesh(core_axis_name="c", subcore_axis_name="s",
                                  num_cores=NC, num_subcores=NS)
    gather = pl.kernel(
        _gather_body,
        out_type=[jax.ShapeDtypeStruct((B, RPB * D), jnp.float32),
                  jax.ShapeDtypeStruct((B, RPB * D), jnp.float32)],
        mesh=mesh,
        compiler_params=pltpu.CompilerParams(needs_layout_passes=False),
        scratch_types=(
            [pltpu.VMEM((WV,), jnp.int32)] * 8 +
            [pltpu.VMEM((WV, RPB * D), jnp.float32)] * 4 +
            [pltpu.SemaphoreType.DMA] * 2
        ),
    )
    ublocks, iblocks = gather(users, items, ut, it)

    return pl.pallas_call(
        _dense_body,
        grid=(TCG,),
        in_specs=[
            pl.BlockSpec((TCB, RPB * D), lambda g: (g, 0)),
            pl.BlockSpec((TCB, RPB * D), lambda g: (g, 0)),
            pl.BlockSpec((TCB, 1), lambda g: (g, 0)),
            pl.BlockSpec((TCB, 1), lambda g: (g, 0)),
            pl.BlockSpec((1, D), lambda g: (0, 0)),
            pl.BlockSpec((1, 1), lambda g: (0, 0)),
        ],
        out_specs=pl.BlockSpec((TCB, 1), lambda g: (g, 0)),
        out_shape=jax.ShapeDtypeStruct((B, 1), jnp.float32),
    )(ublocks, iblocks, users2, items2, wrow, brow)


def kernel(items, users, user_table, item_table, W, b):
    ut = user_table.reshape(R // RPB, RPB * D)
    it = item_table.reshape(R // RPB, RPB * D)
    users = users.astype(jnp.int32)
    items = items.astype(jnp.int32)
    return _gmf(users, items, ut, it,
                users.reshape(B, 1), items.reshape(B, 1),
                W.reshape(1, D).astype(jnp.float32),
                b.reshape(1, 1).astype(jnp.float32))


# speedup vs baseline: 1.0239x; 1.0239x over previous
import jax
import jax.numpy as jnp
from jax import lax
from jax.experimental import pallas as pl
from jax.experimental.pallas import tpu as pltpu
from jax.experimental.pallas import tpu_sc as plsc

R = 1000000
D = 32
B = 16384
RPB = 128 // D

NC = 2
NS = 16
L = 16
NW = NC * NS
BPW = B // NW
H = BPW // 2
NKH = H // L


def _gmf_body(users_h, items_h, ut_h, it_h, ws_h, bs_h, out_h,
              uidx, iidx, qu, qi, gu, gi, wsv, bsv, outv, sem):
    wid = lax.axis_index("s") * NC + lax.axis_index("c")
    base = wid * BPW

    pltpu.sync_copy(users_h.at[pl.ds(base, BPW)], uidx)
    pltpu.sync_copy(items_h.at[pl.ds(base, BPW)], iidx)
    pltpu.sync_copy(ws_h, wsv)
    pltpu.sync_copy(bs_h, bsv)

    bias = bsv[...]
    ivec = lax.iota(jnp.int32, L)

    def half(h, carry):
        hb = h * H
        def mkq(k, c):
            jj = pl.ds(hb + k * L, L)
            qu[pl.ds(k * L, L)] = lax.shift_right_logical(uidx[jj], 2)
            qi[pl.ds(k * L, L)] = lax.shift_right_logical(iidx[jj], 2)
            return c

        lax.fori_loop(0, NKH, mkq, 0)

        cu = pltpu.async_copy(ut_h.at[qu], gu, sem)
        ci = pltpu.async_copy(it_h.at[qi], gi, sem)
        cu.wait()
        ci.wait()

        def chunk(k, c):
            jj = pl.ds(hb + k * L, L)
            bvec = ivec + k * L
            uoff = lax.shift_left(uidx[jj] & 3, 5)
            ioff = lax.shift_left(iidx[jj] & 3, 5)
            acc = bias
            for d in range(D):
                u = plsc.load_gather(gu, [bvec, uoff + d])
                i = plsc.load_gather(gi, [bvec, ioff + d])
                acc = acc + wsv[d] * (u * i)
            outv[pl.ds(hb + k * L, L)] = 1.0 / (1.0 + jnp.exp(-acc))
            return c

        lax.fori_loop(0, NKH, chunk, 0)
        return carry

    lax.fori_loop(0, 2, half, 0)

    pltpu.sync_copy(outv, out_h.at[pl.ds(base, BPW)])


@jax.jit
def _gmf(users, items, ut, it, wsplat, bsplat):
    mesh = plsc.VectorSubcoreMesh(core_axis_name="c", subcore_axis_name="s",
                                  num_cores=NC, num_subcores=NS)
    run = pl.kernel(
        _gmf_body,
        out_type=jax.ShapeDtypeStruct((B,), jnp.float32),
        mesh=mesh,
        compiler_params=pltpu.CompilerParams(needs_layout_passes=False),
        scratch_types=[
            pltpu.VMEM((BPW,), jnp.int32),
            pltpu.VMEM((BPW,), jnp.int32),
            pltpu.VMEM((H,), jnp.int32),
            pltpu.VMEM((H,), jnp.int32),
            pltpu.VMEM((H, 128), jnp.float32),
            pltpu.VMEM((H, 128), jnp.float32),
            pltpu.VMEM((D, L), jnp.float32),
            pltpu.VMEM((L,), jnp.float32),
            pltpu.VMEM((BPW,), jnp.float32),
            pltpu.SemaphoreType.DMA,
        ],
    )
    return run(users, items, ut, it, wsplat, bsplat)


def kernel(items, users, user_table, item_table, W, b):
    ut = user_table.reshape(R // RPB, D * RPB)
    it = item_table.reshape(R // RPB, D * RPB)
    wsplat = jnp.broadcast_to(W.reshape(D, 1), (D, L)).astype(jnp.float32)
    bsplat = jnp.broadcast_to(b.reshape(1), (L,)).astype(jnp.float32)
    out = _gmf(users.astype(jnp.int32), items.astype(jnp.int32),
               ut, it, wsplat, bsplat)
    return out.reshape(B, 1)
